# baseline (speedup 1.0000x reference)
import jax
import jax.numpy as jnp
from jax.experimental import pallas as pl
from jax.experimental.pallas import tpu as pltpu


def _tiny_block(x_ref, w1_ref, out_ref):
    out_ref[...] = w1_ref[:8, :32] * 2.0


def kernel(x, W1, b1, W2, b2):
    n, d_in = x.shape
    d_out = W2.shape[0]
    out = pl.pallas_call(
        _tiny_block,
        grid=(1,),
        in_specs=[
            pl.BlockSpec(memory_space=pl.ANY),
            pl.BlockSpec((8, W1.shape[1]), lambda i: (0, 0)),
        ],
        out_specs=pl.BlockSpec((8, d_out), lambda i: (0, 0)),
        out_shape=jax.ShapeDtypeStruct((8, d_out), jnp.float32),
    )(x, W1)
    return out


# speedup vs baseline: 2.1172x; 2.1172x over previous
import jax
import jax.numpy as jnp
from jax.experimental import pallas as pl
from jax.experimental.pallas import tpu as pltpu

BLOCK = 16384


def _mlp_t_block(x_ref, w1_ref, b1_ref, w2_ref, b2_ref, out_ref):
    xb = x_ref[...].astype(jnp.bfloat16)
    h = jnp.tanh(
        jnp.dot(w1_ref[...], xb, preferred_element_type=jnp.float32) + b1_ref[...]
    )
    out_ref[...] = (
        jnp.dot(
            w2_ref[...], h.astype(jnp.bfloat16), preferred_element_type=jnp.float32
        )
        + b2_ref[...]
    )


def kernel(x, W1, b1, W2, b2):
    n, d_in = x.shape
    hidden = W1.shape[0]
    d_out = W2.shape[0]

    xt = x.T
    w1b = W1.astype(jnp.bfloat16)
    w2b = W2.astype(jnp.bfloat16)
    b1c = b1.reshape(hidden, 1)
    b2c = b2.reshape(d_out, 1)

    grid = (n // BLOCK,)
    out_t = pl.pallas_call(
        _mlp_t_block,
        grid=grid,
        in_specs=[
            pl.BlockSpec((d_in, BLOCK), lambda i: (0, i)),
            pl.BlockSpec((hidden, d_in), lambda i: (0, 0)),
            pl.BlockSpec((hidden, 1), lambda i: (0, 0)),
            pl.BlockSpec((d_out, hidden), lambda i: (0, 0)),
            pl.BlockSpec((d_out, 1), lambda i: (0, 0)),
        ],
        out_specs=pl.BlockSpec((d_out, BLOCK), lambda i: (0, i)),
        out_shape=jax.ShapeDtypeStruct((d_out, n), jnp.float32),
        compiler_params=pltpu.CompilerParams(
            dimension_semantics=("parallel",),
        ),
    )(xt, w1b, b1c, w2b, b2c)
    return out_t.T


# speedup vs baseline: 2.2679x; 1.0712x over previous
import jax
import jax.numpy as jnp
from jax.experimental import pallas as pl
from jax.experimental.pallas import tpu as pltpu

BLOCK = 16384


def _mlp_t_block(x_ref, w1_ref, b1_ref, w2_ref, b2_ref, out_ref):
    xb = x_ref[...].astype(jnp.bfloat16)
    w1b = w1_ref[...].astype(jnp.bfloat16)
    w2b = w2_ref[...].astype(jnp.bfloat16)
    h = jnp.tanh(
        jnp.dot(w1b, xb, preferred_element_type=jnp.float32) + b1_ref[...]
    )
    out_ref[...] = (
        jnp.dot(w2b, h.astype(jnp.bfloat16), preferred_element_type=jnp.float32)
        + b2_ref[...]
    )


def kernel(x, W1, b1, W2, b2):
    n, d_in = x.shape
    hidden = W1.shape[0]
    d_out = W2.shape[0]

    xt = x.T
    b1c = b1.reshape(hidden, 1)
    b2c = b2.reshape(d_out, 1)

    grid = (n // BLOCK,)
    out_t = pl.pallas_call(
        _mlp_t_block,
        grid=grid,
        in_specs=[
            pl.BlockSpec((d_in, BLOCK), lambda i: (0, i)),
            pl.BlockSpec((hidden, d_in), lambda i: (0, 0)),
            pl.BlockSpec((hidden, 1), lambda i: (0, 0)),
            pl.BlockSpec((d_out, hidden), lambda i: (0, 0)),
            pl.BlockSpec((d_out, 1), lambda i: (0, 0)),
        ],
        out_specs=pl.BlockSpec((d_out, BLOCK), lambda i: (0, i)),
        out_shape=jax.ShapeDtypeStruct((d_out, n), jnp.float32),
        compiler_params=pltpu.CompilerParams(
            dimension_semantics=("parallel",),
        ),
    )(xt, W1, b1c, W2, b2c)
    return out_t.T


# speedup vs baseline: 2.3879x; 1.0529x over previous
import jax
import jax.numpy as jnp
from jax.experimental import pallas as pl
from jax.experimental.pallas import tpu as pltpu

BLOCK = 32768


def _mlp_t_block(x_ref, w1_ref, b1_ref, w2_ref, b2_ref, out_ref):
    xb = x_ref[...].astype(jnp.bfloat16)
    w1b = w1_ref[...].astype(jnp.bfloat16)
    w2b = w2_ref[...].astype(jnp.bfloat16)
    h = jnp.tanh(
        jnp.dot(w1b, xb, preferred_element_type=jnp.float32) + b1_ref[...]
    )
    out_ref[...] = (
        jnp.dot(w2b, h.astype(jnp.bfloat16), preferred_element_type=jnp.float32)
        + b2_ref[...]
    )


def kernel(x, W1, b1, W2, b2):
    n, d_in = x.shape
    hidden = W1.shape[0]
    d_out = W2.shape[0]

    xt = x.T
    b1c = b1.reshape(hidden, 1)
    b2c = b2.reshape(d_out, 1)

    grid = (n // BLOCK,)
    out_t = pl.pallas_call(
        _mlp_t_block,
        grid=grid,
        in_specs=[
            pl.BlockSpec((d_in, BLOCK), lambda i: (0, i)),
            pl.BlockSpec((hidden, d_in), lambda i: (0, 0)),
            pl.BlockSpec((hidden, 1), lambda i: (0, 0)),
            pl.BlockSpec((d_out, hidden), lambda i: (0, 0)),
            pl.BlockSpec((d_out, 1), lambda i: (0, 0)),
        ],
        out_specs=pl.BlockSpec((d_out, BLOCK), lambda i: (0, i)),
        out_shape=jax.ShapeDtypeStruct((d_out, n), jnp.float32),
        compiler_params=pltpu.CompilerParams(
            dimension_semantics=("parallel",),
        ),
    )(xt, W1, b1c, W2, b2c)
    return out_t.T


# speedup vs baseline: 2.5702x; 1.0764x over previous
import jax
import jax.numpy as jnp
from jax.experimental import pallas as pl
from jax.experimental.pallas import tpu as pltpu

BLOCK = 32768


def _mlp_t_block(x_ref, w1_ref, b1_ref, w2_ref, b2_ref, out_ref):
    xb = x_ref[...].astype(jnp.bfloat16)
    w1b = w1_ref[...].astype(jnp.bfloat16)
    w2b = w2_ref[...].astype(jnp.bfloat16)
    b1col = b1_ref[...].T
    b2col = b2_ref[...].T
    h = jnp.tanh(jnp.dot(w1b, xb, preferred_element_type=jnp.float32) + b1col)
    out_ref[...] = (
        jnp.dot(w2b, h.astype(jnp.bfloat16), preferred_element_type=jnp.float32)
        + b2col
    )


def kernel(x, W1, b1, W2, b2):
    n, d_in = x.shape
    hidden = W1.shape[0]
    d_out = W2.shape[0]

    xt = x.T
    b1r = b1.reshape(1, hidden)
    b2r = b2.reshape(1, d_out)

    grid = (n // BLOCK,)
    out_t = pl.pallas_call(
        _mlp_t_block,
        grid=grid,
        in_specs=[
            pl.BlockSpec((d_in, BLOCK), lambda i: (0, i)),
            pl.BlockSpec((hidden, d_in), lambda i: (0, 0)),
            pl.BlockSpec((1, hidden), lambda i: (0, 0)),
            pl.BlockSpec((d_out, hidden), lambda i: (0, 0)),
            pl.BlockSpec((1, d_out), lambda i: (0, 0)),
        ],
        out_specs=pl.BlockSpec((d_out, BLOCK), lambda i: (0, i)),
        out_shape=jax.ShapeDtypeStruct((d_out, n), jnp.float32),
        compiler_params=pltpu.CompilerParams(
            dimension_semantics=("parallel",),
        ),
    )(xt, W1, b1r, W2, b2r)
    return out_t.T


# speedup vs baseline: 2.6675x; 1.0378x over previous
import jax
import jax.numpy as jnp
from jax.experimental import pallas as pl
from jax.experimental.pallas import tpu as pltpu

BLOCK = 65536


def _mlp_t_block(x_ref, w1_ref, b1_ref, w2_ref, b2_ref, out_ref):
    xb = x_ref[...].astype(jnp.bfloat16)
    w1b = w1_ref[...].astype(jnp.bfloat16)
    w2b = w2_ref[...].astype(jnp.bfloat16)
    b1col = b1_ref[...].T
    b2col = b2_ref[...].T
    h = jnp.tanh(jnp.dot(w1b, xb, preferred_element_type=jnp.float32) + b1col)
    out_ref[...] = (
        jnp.dot(w2b, h.astype(jnp.bfloat16), preferred_element_type=jnp.float32)
        + b2col
    )


def kernel(x, W1, b1, W2, b2):
    n, d_in = x.shape
    hidden = W1.shape[0]
    d_out = W2.shape[0]

    xt = x.T
    b1r = b1.reshape(1, hidden)
    b2r = b2.reshape(1, d_out)

    grid = (n // BLOCK,)
    out_t = pl.pallas_call(
        _mlp_t_block,
        grid=grid,
        in_specs=[
            pl.BlockSpec((d_in, BLOCK), lambda i: (0, i)),
            pl.BlockSpec((hidden, d_in), lambda i: (0, 0)),
            pl.BlockSpec((1, hidden), lambda i: (0, 0)),
            pl.BlockSpec((d_out, hidden), lambda i: (0, 0)),
            pl.BlockSpec((1, d_out), lambda i: (0, 0)),
        ],
        out_specs=pl.BlockSpec((d_out, BLOCK), lambda i: (0, i)),
        out_shape=jax.ShapeDtypeStruct((d_out, n), jnp.float32),
        compiler_params=pltpu.CompilerParams(
            dimension_semantics=("parallel",),
            vmem_limit_bytes=100 * 1024 * 1024,
        ),
    )(xt, W1, b1r, W2, b2r)
    return out_t.T
